# baseline (speedup 1.0000x reference)
import functools

import jax
import jax.numpy as jnp
from jax import lax
from jax.experimental import pallas as pl
from jax.experimental.pallas import tpu as pltpu
from jax.experimental.pallas import tpu_sc as plsc

NC = 1
NS = 16
NW = NC * NS
LANES = 16

EC = 64
EC1 = 2048
CORE0_SHARE = 40


def _make_sc_kernel(n_pad, e_pad, d):
    slc = n_pad // NS
    nr = n_pad // d
    rpt = 8
    ntile_deg = nr // rpt
    te_all = e_pad // NS
    nch_all = te_all // EC
    nch0 = (nch_all * CORE0_SHARE // 40) // 4 * 4
    nch1 = nch_all - nch0
    assert nch0 % 4 == 0 and nch1 % 4 == 0 and nch0 >= 4
    assert NC == 2 or nch1 == 0
    te0 = nch0 * EC
    te1c = nch1 * EC
    te1 = e_pad // NS
    n_chunks1 = te1 // EC1
    nvec = n_pad // LANES

    mesh = plsc.VectorSubcoreMesh(core_axis_name="c", subcore_axis_name="s",
                                  num_cores=NC, num_subcores=NS)

    @functools.partial(
        pl.kernel,
        out_type=[
            jax.ShapeDtypeStruct((NC * n_pad, d), jnp.float32),
            jax.ShapeDtypeStruct((NC * nr, d), jnp.float32),
            jax.ShapeDtypeStruct((NW * nr, d), jnp.float32),
        ],
        mesh=mesh,
        compiler_params=pltpu.CompilerParams(needs_layout_passes=False),
        scratch_types=[
            pltpu.VMEM_SHARED((n_pad, d), jnp.float32),
            pltpu.VMEM_SHARED((nr, d), jnp.float32),
            pltpu.VMEM((nr, d), jnp.float32),
            pltpu.VMEM((n_pad,), jnp.float32),
            pltpu.VMEM((EC1,), jnp.int32),
            [pltpu.VMEM((EC,), jnp.int32) for _ in range(4)],
            [pltpu.VMEM((EC,), jnp.int32) for _ in range(4)],
            [pltpu.VMEM((EC, d), jnp.float32) for _ in range(2)],
            pltpu.VMEM((nr,), jnp.int32),
            pltpu.SemaphoreType.DMA,
            [pltpu.SemaphoreType.DMA for _ in range(2)],
            [pltpu.SemaphoreType.DMA for _ in range(2)],
        ],
    )
    def sc_kernel(h_hbm, src_hbm, dst_hbm,
                  agg_out, deg_out, omega_out,
                  agg_sh, deg_sh,
                  hist_loc, inv_loc,
                  dst1_buf, srcs, dsts, rows, iota_buf,
                  sem, gsem, ssem):
        cid = lax.axis_index("c")
        sid = lax.axis_index("s")
        wid = cid * NS + sid

        zero16 = jnp.zeros((LANES,), jnp.float32)
        one16 = jnp.full((LANES,), 1.0, jnp.float32)
        lo_mask = jnp.full((LANES,), d - 1, jnp.int32)
        hi_shift = jnp.full((LANES,), d.bit_length() - 1, jnp.int32)

        def zero_hist(_, __):
            def zr(r, _):
                def zc(c, _):
                    hist_loc[r, pl.ds(c * LANES, LANES)] = zero16
                    return 0
                lax.fori_loop(0, d // LANES, zc, 0)
                return 0
            lax.fori_loop(0, nr, zr, 0)
            return 0

        def zrows(i, _):
            def zrow16(j, _):
                rows[0][i, pl.ds(j * LANES, LANES)] = zero16
                return 0
            lax.fori_loop(0, d // LANES, zrow16, 0)
            return 0
        lax.fori_loop(0, EC, zrows, 0)
        for c in range(slc // EC):
            pltpu.sync_copy(rows[0], agg_sh.at[pl.ds(sid * slc + c * EC, EC)])
        @pl.when(sid < ntile_deg)
        def _():
            pltpu.sync_copy(rows[0].at[pl.ds(0, rpt)],
                            deg_sh.at[pl.ds(sid * rpt, rpt)])
        zero_hist(0, 0)
        def fill_iota(i, _):
            iota_buf[pl.ds(i * LANES, LANES)] = (
                lax.broadcasted_iota(jnp.int32, (LANES,), 0) + i * LANES)
            return 0
        lax.fori_loop(0, nr // LANES, fill_iota, 0)
        plsc.subcore_barrier()

        def deg_chunk(i, _):
            base = sid * te1 + i * EC1
            pltpu.sync_copy(dst_hbm.at[pl.ds(base, EC1)], dst1_buf)
            def hist16(j, _):
                idx = dst1_buf[pl.ds(j * LANES, LANES)]
                hi = jax.lax.shift_right_logical(idx, hi_shift)
                lo = jax.lax.bitwise_and(idx, lo_mask)
                plsc.addupdate_scatter(hist_loc, [hi, lo], one16)
                return 0
            lax.fori_loop(0, EC1 // LANES, hist16, 0)
            return 0
        lax.fori_loop(0, n_chunks1, deg_chunk, 0)
        pltpu.sync_copy(hist_loc, deg_sh.at[iota_buf], add=True)
        plsc.subcore_barrier()

        assert nr <= 2 * EC
        parts = [(0, min(EC, nr))]
        if nr > EC:
            parts.append((EC, nr - EC))
        for bi, (off, take) in enumerate(parts):
            pltpu.sync_copy(deg_sh.at[pl.ds(off, take)],
                            rows[bi].at[pl.ds(0, take)])
        def make_invr(buf, row_off):
            def invr(r, _):
                def invc(c, _):
                    dv = buf[r, pl.ds(c * LANES, LANES)]
                    inv_loc[pl.ds((row_off + r) * d + c * LANES, LANES)] = \
                        1.0 / jnp.maximum(dv, 1.0)
                    return 0
                lax.fori_loop(0, d // LANES, invc, 0)
                return 0
            return invr
        lax.fori_loop(0, parts[0][1], make_invr(rows[0], 0), 0)
        if len(parts) > 1:
            lax.fori_loop(0, parts[1][1], make_invr(rows[1], parts[0][1]), 0)
        zero_hist(0, 0)

        n_chunks = jnp.where(cid == 0, nch0, nch1)
        ebase = jnp.where(cid == 0, sid * te0, NS * te0 + sid * te1c)

        def load_idx(i, k):
            pltpu.sync_copy(src_hbm.at[pl.ds(ebase + i * EC, EC)], srcs[k])
            pltpu.sync_copy(dst_hbm.at[pl.ds(ebase + i * EC, EC)], dsts[k])

        def omega_chunk(k):
            def om16(j, _):
                dv = dsts[k][pl.ds(j * LANES, LANES)]
                w = plsc.load_gather(inv_loc, [dv])
                sv = srcs[k][pl.ds(j * LANES, LANES)]
                hi = jax.lax.shift_right_logical(sv, hi_shift)
                lo = jax.lax.bitwise_and(sv, lo_mask)
                plsc.addupdate_scatter(hist_loc, [hi, lo], w)
                return 0
            lax.fori_loop(0, EC // LANES, om16, 0)

        load_idx(0, 0)
        load_idx(1, 1)
        pltpu.async_copy(h_hbm.at[srcs[0]], rows[0], gsem[0])

        def group(g, _):
            for k in range(4):
                i = g * 4 + k
                b = k % 2
                @pl.when(i > 0)
                def _():
                    pltpu.make_async_copy(
                        rows[1 - b], agg_sh.at[dsts[(k - 1) % 4]],
                        ssem[1 - b]).wait()
                @pl.when(i + 1 < n_chunks)
                def _():
                    pltpu.async_copy(h_hbm.at[srcs[(k + 1) % 4]],
                                     rows[1 - b], gsem[1 - b])
                @pl.when(i + 2 < n_chunks)
                def _():
                    load_idx(i + 2, (k + 2) % 4)
                pltpu.make_async_copy(h_hbm.at[srcs[k]], rows[b],
                                      gsem[b]).wait()
                pltpu.async_copy(rows[b], agg_sh.at[dsts[k]], ssem[b],
                                 add=True)
                omega_chunk(k)
            return 0
        lax.fori_loop(0, n_chunks // 4, group, 0)
        pltpu.make_async_copy(rows[1], agg_sh.at[dsts[3]], ssem[1]).wait()
        plsc.subcore_barrier()

        pltpu.sync_copy(agg_sh.at[pl.ds(sid * slc, slc)],
                        agg_out.at[pl.ds(cid * n_pad + sid * slc, slc)])
        @pl.when(sid < ntile_deg)
        def _():
            pltpu.sync_copy(deg_sh.at[pl.ds(sid * rpt, rpt)],
                            deg_out.at[pl.ds(cid * nr + sid * rpt, rpt)])
        pltpu.sync_copy(hist_loc, omega_out.at[pl.ds(wid * nr, nr)])

    return sc_kernel


def _make_tc_kernel(n, n_pad, d, blk):
    nb = n_pad // blk

    def tc_body(h_ref, agg_a_ref, agg_b_ref, deg_ref, om_ref,
                ws1_ref, wn1_ref, b1_ref, ws2_ref, wn2_ref, b2_ref,
                wc_ref, bc_ref, out_ref, s_self_acc, s_neigh_acc):
        i = pl.program_id(0)

        @pl.when(i == 0)
        def _():
            s_self_acc[...] = jnp.zeros_like(s_self_acc)
            s_neigh_acc[...] = jnp.zeros_like(s_neigh_acc)

        inv = 1.0 / jnp.maximum(deg_ref[...], 1.0)
        mean1 = (agg_a_ref[...] + agg_b_ref[...]) * inv
        h1 = jnp.maximum(
            jnp.dot(h_ref[...], ws1_ref[...], preferred_element_type=jnp.float32)
            + jnp.dot(mean1, wn1_ref[...], preferred_element_type=jnp.float32)
            + b1_ref[...], 0.0)
        rows = lax.broadcasted_iota(jnp.int32, (blk, 1), 0) + i * blk
        mask = (rows < n).astype(jnp.float32)
        omega = jnp.sum(om_ref[...], axis=1, keepdims=True)
        s_self_acc[...] += jnp.sum(h1 * mask, axis=0, keepdims=True)
        s_neigh_acc[...] += jnp.sum(h1 * (omega * mask), axis=0, keepdims=True)

        @pl.when(i == nb - 1)
        def _():
            hg = (jnp.dot(s_self_acc[...], ws2_ref[...],
                          preferred_element_type=jnp.float32)
                  + jnp.dot(s_neigh_acc[...], wn2_ref[...],
                            preferred_element_type=jnp.float32)) * (1.0 / n) \
                 + b2_ref[...]
            out_ref[...] = jnp.dot(hg, wc_ref[...],
                                   preferred_element_type=jnp.float32) + bc_ref[...]

    row_spec = lambda w: pl.BlockSpec((blk, w), lambda i: (i, 0))
    full_spec = lambda r, c: pl.BlockSpec((r, c), lambda i: (0, 0))

    return pl.pallas_call(
        tc_body,
        grid=(nb,),
        in_specs=[
            row_spec(d),
            row_spec(d),
            row_spec(d),
            row_spec(1),
            row_spec(NW),
            full_spec(d, d),
            full_spec(d, d),
            full_spec(1, d),
            full_spec(d, d),
            full_spec(d, d),
            full_spec(1, d),
            full_spec(d, d),
            full_spec(1, d),
        ],
        out_specs=pl.BlockSpec((1, d), lambda i: (0, 0)),
        out_shape=jax.ShapeDtypeStruct((1, d), jnp.float32),
        scratch_shapes=[
            pltpu.VMEM((1, d), jnp.float32),
            pltpu.VMEM((1, d), jnp.float32),
        ],
    )


def kernel(h, edge_index, W_self1, W_neigh1, b1, W_self2, W_neigh2, b2,
           W_cls, b_cls):
    n, d = h.shape
    e = edge_index.shape[1]
    n_cls = W_cls.shape[1]

    n_pad = ((n + 2047) // 2048) * 2048
    e_pad = ((e + NS * EC1 - 1) // (NS * EC1)) * (NS * EC1)

    src = edge_index[0].astype(jnp.int32)
    dst = edge_index[1].astype(jnp.int32)
    h_pad = jnp.pad(h, ((0, n_pad - n), (0, 0)))
    fill = jnp.full((e_pad - e,), n_pad - 1, jnp.int32)
    src_p = jnp.concatenate([src, fill])
    dst_p = jnp.concatenate([dst, fill])

    agg2, deg2, om = _make_sc_kernel(n_pad, e_pad, d)(h_pad, src_p, dst_p)

    nr = n_pad // d
    agg_a = agg2[:n_pad]
    agg_b = agg2[n_pad:] if NC == 2 else jnp.zeros_like(agg_a)
    deg_col = deg2[:nr].reshape(n_pad, 1)
    om_t = om.reshape(NW, n_pad).T

    wc_pad = jnp.pad(W_cls, ((0, 0), (0, d - n_cls)))
    bc_pad = jnp.pad(b_cls, (0, d - n_cls)).reshape(1, d)

    out = _make_tc_kernel(n, n_pad, d, 2048)(
        h_pad, agg_a, agg_b, deg_col, om_t,
        W_self1, W_neigh1, b1.reshape(1, d),
        W_self2, W_neigh2, b2.reshape(1, d),
        wc_pad, bc_pad)
    return out[:, :n_cls]


# speedup vs baseline: 1.6148x; 1.6148x over previous
import functools

import jax
import jax.numpy as jnp
from jax import lax
from jax.experimental import pallas as pl
from jax.experimental.pallas import tpu as pltpu
from jax.experimental.pallas import tpu_sc as plsc

NC = 2
NS = 16
NW = NC * NS
LANES = 16

EC = 64
EC1 = 2048
CORE0_SHARE = 34


def _make_sc_kernel(n_pad, e_pad, d):
    slc = n_pad // NS
    nr = n_pad // d
    rpt = 8
    ntile_deg = nr // rpt
    te_all = e_pad // NS
    nch_all = te_all // EC
    nch0 = (nch_all * CORE0_SHARE // 40) // 4 * 4
    nch1 = nch_all - nch0
    assert nch0 % 4 == 0 and nch1 % 4 == 0 and nch0 >= 4
    assert NC == 2 or nch1 == 0
    te0 = nch0 * EC
    te1c = nch1 * EC
    te1 = e_pad // NS
    n_chunks1 = te1 // EC1
    nvec = n_pad // LANES

    mesh = plsc.VectorSubcoreMesh(core_axis_name="c", subcore_axis_name="s",
                                  num_cores=NC, num_subcores=NS)

    @functools.partial(
        pl.kernel,
        out_type=[
            jax.ShapeDtypeStruct((NC * n_pad, d), jnp.float32),
            jax.ShapeDtypeStruct((NC * nr, d), jnp.float32),
            jax.ShapeDtypeStruct((NW * nr, d), jnp.float32),
        ],
        mesh=mesh,
        compiler_params=pltpu.CompilerParams(needs_layout_passes=False),
        scratch_types=[
            pltpu.VMEM_SHARED((n_pad, d), jnp.float32),
            pltpu.VMEM_SHARED((nr, d), jnp.float32),
            pltpu.VMEM((nr, d), jnp.float32),
            pltpu.VMEM((n_pad,), jnp.float32),
            pltpu.VMEM((EC1,), jnp.int32),
            [pltpu.VMEM((EC,), jnp.int32) for _ in range(4)],
            [pltpu.VMEM((EC,), jnp.int32) for _ in range(4)],
            [pltpu.VMEM((EC, d), jnp.float32) for _ in range(2)],
            pltpu.VMEM((nr,), jnp.int32),
            pltpu.SemaphoreType.DMA,
            [pltpu.SemaphoreType.DMA for _ in range(2)],
            [pltpu.SemaphoreType.DMA for _ in range(2)],
        ],
    )
    def sc_kernel(h_hbm, src_hbm, dst_hbm,
                  agg_out, deg_out, omega_out,
                  agg_sh, deg_sh,
                  hist_loc, inv_loc,
                  dst1_buf, srcs, dsts, rows, iota_buf,
                  sem, gsem, ssem):
        cid = lax.axis_index("c")
        sid = lax.axis_index("s")
        wid = cid * NS + sid

        zero16 = jnp.zeros((LANES,), jnp.float32)
        one16 = jnp.full((LANES,), 1.0, jnp.float32)
        lo_mask = jnp.full((LANES,), d - 1, jnp.int32)
        hi_shift = jnp.full((LANES,), d.bit_length() - 1, jnp.int32)

        def zero_hist(_, __):
            def zr(r, _):
                def zc(c, _):
                    hist_loc[r, pl.ds(c * LANES, LANES)] = zero16
                    return 0
                lax.fori_loop(0, d // LANES, zc, 0)
                return 0
            lax.fori_loop(0, nr, zr, 0)
            return 0

        def zrows(i, _):
            def zrow16(j, _):
                rows[0][i, pl.ds(j * LANES, LANES)] = zero16
                return 0
            lax.fori_loop(0, d // LANES, zrow16, 0)
            return 0
        lax.fori_loop(0, EC, zrows, 0)
        for c in range(slc // EC):
            pltpu.sync_copy(rows[0], agg_sh.at[pl.ds(sid * slc + c * EC, EC)])
        @pl.when(sid < ntile_deg)
        def _():
            pltpu.sync_copy(rows[0].at[pl.ds(0, rpt)],
                            deg_sh.at[pl.ds(sid * rpt, rpt)])
        zero_hist(0, 0)
        def fill_iota(i, _):
            iota_buf[pl.ds(i * LANES, LANES)] = (
                lax.broadcasted_iota(jnp.int32, (LANES,), 0) + i * LANES)
            return 0
        lax.fori_loop(0, nr // LANES, fill_iota, 0)
        plsc.subcore_barrier()

        def deg_chunk(i, _):
            base = sid * te1 + i * EC1
            pltpu.sync_copy(dst_hbm.at[pl.ds(base, EC1)], dst1_buf)
            def hist16(j, _):
                idx = dst1_buf[pl.ds(j * LANES, LANES)]
                hi = jax.lax.shift_right_logical(idx, hi_shift)
                lo = jax.lax.bitwise_and(idx, lo_mask)
                plsc.addupdate_scatter(hist_loc, [hi, lo], one16)
                return 0
            lax.fori_loop(0, EC1 // LANES, hist16, 0)
            return 0
        lax.fori_loop(0, n_chunks1, deg_chunk, 0)
        pltpu.sync_copy(hist_loc, deg_sh.at[iota_buf], add=True)
        plsc.subcore_barrier()

        assert nr <= 2 * EC
        parts = [(0, min(EC, nr))]
        if nr > EC:
            parts.append((EC, nr - EC))
        for bi, (off, take) in enumerate(parts):
            pltpu.sync_copy(deg_sh.at[pl.ds(off, take)],
                            rows[bi].at[pl.ds(0, take)])
        def make_invr(buf, row_off):
            def invr(r, _):
                def invc(c, _):
                    dv = buf[r, pl.ds(c * LANES, LANES)]
                    inv_loc[pl.ds((row_off + r) * d + c * LANES, LANES)] = \
                        1.0 / jnp.maximum(dv, 1.0)
                    return 0
                lax.fori_loop(0, d // LANES, invc, 0)
                return 0
            return invr
        lax.fori_loop(0, parts[0][1], make_invr(rows[0], 0), 0)
        if len(parts) > 1:
            lax.fori_loop(0, parts[1][1], make_invr(rows[1], parts[0][1]), 0)
        zero_hist(0, 0)

        n_chunks = jnp.where(cid == 0, nch0, nch1)
        ebase = jnp.where(cid == 0, sid * te0, NS * te0 + sid * te1c)

        def load_idx(i, k):
            pltpu.sync_copy(src_hbm.at[pl.ds(ebase + i * EC, EC)], srcs[k])
            pltpu.sync_copy(dst_hbm.at[pl.ds(ebase + i * EC, EC)], dsts[k])

        def omega_chunk(k):
            def om16(j, _):
                dv = dsts[k][pl.ds(j * LANES, LANES)]
                w = plsc.load_gather(inv_loc, [dv])
                sv = srcs[k][pl.ds(j * LANES, LANES)]
                hi = jax.lax.shift_right_logical(sv, hi_shift)
                lo = jax.lax.bitwise_and(sv, lo_mask)
                plsc.addupdate_scatter(hist_loc, [hi, lo], w)
                return 0
            lax.fori_loop(0, EC // LANES, om16, 0)

        load_idx(0, 0)
        load_idx(1, 1)
        pltpu.async_copy(h_hbm.at[srcs[0]], rows[0], gsem[0])

        def group(g, _):
            for k in range(4):
                i = g * 4 + k
                b = k % 2
                @pl.when(i > 0)
                def _():
                    pltpu.make_async_copy(
                        rows[1 - b], agg_sh.at[dsts[(k - 1) % 4]],
                        ssem[1 - b]).wait()
                @pl.when(i + 1 < n_chunks)
                def _():
                    pltpu.async_copy(h_hbm.at[srcs[(k + 1) % 4]],
                                     rows[1 - b], gsem[1 - b])
                @pl.when(i + 2 < n_chunks)
                def _():
                    load_idx(i + 2, (k + 2) % 4)
                pltpu.make_async_copy(h_hbm.at[srcs[k]], rows[b],
                                      gsem[b]).wait()
                pltpu.async_copy(rows[b], agg_sh.at[dsts[k]], ssem[b],
                                 add=True)
                omega_chunk(k)
            return 0
        lax.fori_loop(0, n_chunks // 4, group, 0)
        pltpu.make_async_copy(rows[1], agg_sh.at[dsts[3]], ssem[1]).wait()
        plsc.subcore_barrier()

        pltpu.sync_copy(agg_sh.at[pl.ds(sid * slc, slc)],
                        agg_out.at[pl.ds(cid * n_pad + sid * slc, slc)])
        @pl.when(sid < ntile_deg)
        def _():
            pltpu.sync_copy(deg_sh.at[pl.ds(sid * rpt, rpt)],
                            deg_out.at[pl.ds(cid * nr + sid * rpt, rpt)])
        pltpu.sync_copy(hist_loc, omega_out.at[pl.ds(wid * nr, nr)])

    return sc_kernel


def _make_tc_kernel(n, n_pad, d, blk):
    nb = n_pad // blk

    def tc_body(h_ref, agg_a_ref, agg_b_ref, deg_ref, om_ref,
                ws1_ref, wn1_ref, b1_ref, ws2_ref, wn2_ref, b2_ref,
                wc_ref, bc_ref, out_ref, s_self_acc, s_neigh_acc):
        i = pl.program_id(0)

        @pl.when(i == 0)
        def _():
            s_self_acc[...] = jnp.zeros_like(s_self_acc)
            s_neigh_acc[...] = jnp.zeros_like(s_neigh_acc)

        inv = 1.0 / jnp.maximum(deg_ref[...], 1.0)
        mean1 = (agg_a_ref[...] + agg_b_ref[...]) * inv
        h1 = jnp.maximum(
            jnp.dot(h_ref[...], ws1_ref[...], preferred_element_type=jnp.float32)
            + jnp.dot(mean1, wn1_ref[...], preferred_element_type=jnp.float32)
            + b1_ref[...], 0.0)
        rows = lax.broadcasted_iota(jnp.int32, (blk, 1), 0) + i * blk
        mask = (rows < n).astype(jnp.float32)
        omega = jnp.sum(om_ref[...], axis=1, keepdims=True)
        s_self_acc[...] += jnp.sum(h1 * mask, axis=0, keepdims=True)
        s_neigh_acc[...] += jnp.sum(h1 * (omega * mask), axis=0, keepdims=True)

        @pl.when(i == nb - 1)
        def _():
            hg = (jnp.dot(s_self_acc[...], ws2_ref[...],
                          preferred_element_type=jnp.float32)
                  + jnp.dot(s_neigh_acc[...], wn2_ref[...],
                            preferred_element_type=jnp.float32)) * (1.0 / n) \
                 + b2_ref[...]
            out_ref[...] = jnp.dot(hg, wc_ref[...],
                                   preferred_element_type=jnp.float32) + bc_ref[...]

    row_spec = lambda w: pl.BlockSpec((blk, w), lambda i: (i, 0))
    full_spec = lambda r, c: pl.BlockSpec((r, c), lambda i: (0, 0))

    return pl.pallas_call(
        tc_body,
        grid=(nb,),
        in_specs=[
            row_spec(d),
            row_spec(d),
            row_spec(d),
            row_spec(1),
            row_spec(NW),
            full_spec(d, d),
            full_spec(d, d),
            full_spec(1, d),
            full_spec(d, d),
            full_spec(d, d),
            full_spec(1, d),
            full_spec(d, d),
            full_spec(1, d),
        ],
        out_specs=pl.BlockSpec((1, d), lambda i: (0, 0)),
        out_shape=jax.ShapeDtypeStruct((1, d), jnp.float32),
        scratch_shapes=[
            pltpu.VMEM((1, d), jnp.float32),
            pltpu.VMEM((1, d), jnp.float32),
        ],
    )


def kernel(h, edge_index, W_self1, W_neigh1, b1, W_self2, W_neigh2, b2,
           W_cls, b_cls):
    n, d = h.shape
    e = edge_index.shape[1]
    n_cls = W_cls.shape[1]

    n_pad = ((n + 2047) // 2048) * 2048
    e_pad = ((e + NS * EC1 - 1) // (NS * EC1)) * (NS * EC1)

    src = edge_index[0].astype(jnp.int32)
    dst = edge_index[1].astype(jnp.int32)
    h_pad = jnp.pad(h, ((0, n_pad - n), (0, 0)))
    fill = jnp.full((e_pad - e,), n_pad - 1, jnp.int32)
    src_p = jnp.concatenate([src, fill])
    dst_p = jnp.concatenate([dst, fill])

    agg2, deg2, om = _make_sc_kernel(n_pad, e_pad, d)(h_pad, src_p, dst_p)

    nr = n_pad // d
    agg_a = agg2[:n_pad]
    agg_b = agg2[n_pad:] if NC == 2 else jnp.zeros_like(agg_a)
    deg_col = deg2[:nr].reshape(n_pad, 1)
    om_t = om.reshape(NW, n_pad).T

    wc_pad = jnp.pad(W_cls, ((0, 0), (0, d - n_cls)))
    bc_pad = jnp.pad(b_cls, (0, d - n_cls)).reshape(1, d)

    out = _make_tc_kernel(n, n_pad, d, 2048)(
        h_pad, agg_a, agg_b, deg_col, om_t,
        W_self1, W_neigh1, b1.reshape(1, d),
        W_self2, W_neigh2, b2.reshape(1, d),
        wc_pad, bc_pad)
    return out[:, :n_cls]
